# Initial kernel scaffold; baseline (speedup 1.0000x reference)
import jax
import jax.numpy as jnp
from jax.experimental import pallas as pl


def kernel(x, edge_index, W1, b1, W2, b2, fcW1, fcb1, fcW2, fcb2, fcW3, fcb3):
    raise NotImplementedError("write your pallas kernel here")


# speedup vs baseline: 4.1727x; 4.1727x over previous
import jax
import jax.numpy as jnp
from jax.experimental import pallas as pl

B = 4096
N = 46
NP = 48
CIN = 14
H = 256
EPAD = 256
CHUNK = 8
CR = CHUNK * NP
BT_CONV = 128
NCH = BT_CONV // CHUNK
BT_FC = 256


def _prep_kernel(idx_ref, a_ref, c_ref, avg_ref):
    src = idx_ref[0:1, :]
    dst = idx_ref[1:2, :]
    node = jax.lax.broadcasted_iota(jnp.int32, (NP, EPAD), 0)
    s_t = jnp.where(src == node, 1.0, 0.0)
    d_t = jnp.where(dst == node, 1.0, 0.0)
    deg = jnp.sum(d_t, axis=1, keepdims=True)
    dinv = jnp.where(deg > 0, jax.lax.rsqrt(jnp.maximum(deg, 1e-9)), 0.0)
    dinv_src = jnp.sum(s_t * dinv, axis=0, keepdims=True)
    dinv_dst = jnp.sum(d_t * dinv, axis=0, keepdims=True)
    norm = dinv_src * dinv_dst
    a48 = jax.lax.dot_general(d_t * norm, s_t,
                              (((1,), (1,)), ((), ())),
                              preferred_element_type=jnp.float32)

    r = jax.lax.broadcasted_iota(jnp.int32, (CR, NP), 0)
    i = jax.lax.broadcasted_iota(jnp.int32, (CR, NP), 1)
    p = jnp.where(r % NP == i, 1.0, 0.0)
    t1 = jnp.dot(p, a48, preferred_element_type=jnp.float32)
    t2 = jax.lax.dot_general(t1, p, (((1,), (1,)), ((), ())),
                             preferred_element_type=jnp.float32)
    rr = jax.lax.broadcasted_iota(jnp.int32, (CR, CR), 0)
    ss = jax.lax.broadcasted_iota(jnp.int32, (CR, CR), 1)
    same = (rr // NP) == (ss // NP)
    a_ref[...] = jnp.where(same, t2, 0.0)

    rm = rr % NP
    sm = ss % NP
    eye = jnp.where(rm == sm, 1.0, 0.0)
    sub = jnp.where(sm < N, 1.0 / N, 0.0)
    c_ref[...] = jnp.where(same & (rm < N), eye - sub, 0.0)
    avg_ref[...] = jnp.where(same & (sm < N), 1.0 / N, 0.0)


def _conv_kernel(xs_ref, a_ref, c_ref, avg_ref, w1_ref, b1_ref, w2_ref,
                 b2_ref, out_ref):
    abig = a_ref[...]
    cbig = c_ref[...]
    avg = avg_ref[...]
    w1 = w1_ref[...]
    b1 = b1_ref[...]
    w2 = w2_ref[...]
    b2 = b2_ref[...]
    lane = jax.lax.broadcasted_iota(jnp.int32, (CR, CIN), 1)
    is_coord = lane < 3
    for c in range(NCH):
        xs = xs_ref[c * CR:(c + 1) * CR, :]
        cent = jnp.dot(cbig, xs, preferred_element_type=jnp.float32)
        sq = jnp.where(is_coord, cent * cent, 0.0)
        nrm = jnp.sqrt(jnp.sum(sq, axis=1, keepdims=True))
        scale = jnp.dot(avg, nrm, preferred_element_type=jnp.float32)
        xn = jnp.where(is_coord, cent / (scale + 1e-6), xs)
        g1 = jnp.dot(abig, xn, preferred_element_type=jnp.float32)
        h1 = jax.nn.relu(jnp.dot(g1, w1, preferred_element_type=jnp.float32)
                         + b1)
        hw2 = jnp.dot(h1, w2, preferred_element_type=jnp.float32)
        g2 = jnp.dot(abig, hw2, preferred_element_type=jnp.float32)
        out_ref[c * CR:(c + 1) * CR, :] = jax.nn.relu(g2 + b2)


def _fc_kernel(h_ref, w1_ref, b1_ref, w2_ref, b2_ref, w3_ref, b3_ref,
               out_ref):
    h = h_ref[...]
    y = jax.nn.relu(jnp.dot(h, w1_ref[...],
                            preferred_element_type=jnp.float32) + b1_ref[...])
    y = jax.nn.relu(jnp.dot(y, w2_ref[...],
                            preferred_element_type=jnp.float32) + b2_ref[...])
    out_ref[...] = jnp.dot(y, w3_ref[...],
                           preferred_element_type=jnp.float32) + b3_ref[...]


def _full(shape):
    return pl.BlockSpec(shape, lambda *_: (0,) * len(shape))


@jax.jit
def kernel(x, edge_index, W1, b1, W2, b2, fcW1, fcb1, fcW2, fcb2, fcW3, fcb3):
    xp = jnp.pad(x, ((0, 0), (0, NP - N), (0, 0))).reshape(B * NP, CIN)
    loop = jnp.arange(N, dtype=edge_index.dtype)
    srcf = jnp.concatenate([edge_index[0], loop])
    dstf = jnp.concatenate([edge_index[1], loop])
    idx = jnp.full((8, EPAD), -1, jnp.int32)
    idx = idx.at[0, :srcf.shape[0]].set(srcf.astype(jnp.int32))
    idx = idx.at[1, :dstf.shape[0]].set(dstf.astype(jnp.int32))

    abig, cbig, avg = pl.pallas_call(
        _prep_kernel,
        out_shape=[jax.ShapeDtypeStruct((CR, CR), jnp.float32)] * 3,
        in_specs=[_full((8, EPAD))],
        out_specs=[_full((CR, CR))] * 3,
    )(idx)

    rows = BT_CONV * NP
    h2 = pl.pallas_call(
        _conv_kernel,
        grid=(B // BT_CONV,),
        in_specs=[
            pl.BlockSpec((rows, CIN), lambda i: (i, 0)),
            _full((CR, CR)), _full((CR, CR)), _full((CR, CR)),
            _full((CIN, H)), _full((1, H)), _full((H, H)), _full((1, H)),
        ],
        out_specs=pl.BlockSpec((rows, H), lambda i: (i, 0)),
        out_shape=jax.ShapeDtypeStruct((B * NP, H), jnp.float32),
    )(xp, abig, cbig, avg, W1, b1.reshape(1, H), W2, b2.reshape(1, H))

    h2f = h2.reshape(B, NP * H)
    fcW1p = jnp.pad(fcW1.reshape(N, H, 128), ((0, NP - N), (0, 0), (0, 0)))
    fcW1p = fcW1p.reshape(NP * H, 128)

    out = pl.pallas_call(
        _fc_kernel,
        grid=(B // BT_FC,),
        in_specs=[
            pl.BlockSpec((BT_FC, NP * H), lambda i: (i, 0)),
            _full((NP * H, 128)), _full((1, 128)),
            _full((128, 64)), _full((1, 64)),
            _full((64, 1)), _full((1, 1)),
        ],
        out_specs=pl.BlockSpec((BT_FC, 1), lambda i: (i, 0)),
        out_shape=jax.ShapeDtypeStruct((B, 1), jnp.float32),
    )(h2f, fcW1p, fcb1.reshape(1, 128), fcW2, fcb2.reshape(1, 64),
      fcW3, fcb3.reshape(1, 1))
    return out
